# Initial kernel scaffold; baseline (speedup 1.0000x reference)
import jax
import jax.numpy as jnp
from jax.experimental import pallas as pl


def kernel(x, edge_index, W1, b1, Wg, bg, Wmu, bmu, Wls, bls):
    raise NotImplementedError("write your pallas kernel here")


# speedup vs baseline: 6.5879x; 6.5879x over previous
import functools

import jax
import jax.numpy as jnp
from jax import lax
from jax.experimental import pallas as pl
from jax.experimental.pallas import tpu as pltpu
from jax.experimental.pallas import tpu_sc as plsc

_L = 16
_NTILES = 16
_NCORES = 2
_BATCH = 128
_EB = 48
_NB = 4
_ZR = 16


def _make_deg_dinv(n_nodes, n_edges, npad):
    sl = npad // _NTILES
    ept = n_edges // _NTILES
    nbatch = ept // _BATCH
    erem = ept % _BATCH
    mesh = plsc.VectorSubcoreMesh(core_axis_name="c", subcore_axis_name="s")

    @functools.partial(
        pl.kernel,
        mesh=mesh,
        out_type=jax.ShapeDtypeStruct((npad,), jnp.float32),
        scratch_types=[
            pltpu.VMEM((_BATCH,), jnp.int32),
            pltpu.VMEM((_BATCH,), jnp.float32),
            pltpu.VMEM((sl,), jnp.float32),
            pltpu.VMEM((sl,), jnp.float32),
            pltpu.VMEM_SHARED((npad + _L,), jnp.float32),
            pltpu.SemaphoreType.DMA,
        ],
    )
    def deg_dinv(dst_hbm, deg_hbm, ibuf, ones, degv, dfv, hist, sem):
        cid = lax.axis_index("c")
        sid = lax.axis_index("s")

        @pl.when(cid == 0)
        def _():
            zf = jnp.zeros((_L,), jnp.float32)
            onev = jnp.ones((_L,), jnp.float32)
            lane = lax.iota(jnp.int32, _L)

            def zf_body(i, _):
                degv[pl.ds(i * _L, _L)] = zf
                return 0
            lax.fori_loop(0, sl // _L, zf_body, 0)

            def ones_body(i, _):
                ones[pl.ds(i * _L, _L)] = onev
                return 0
            lax.fori_loop(0, _BATCH // _L, ones_body, 0)

            base = sid * sl
            pltpu.sync_copy(degv, hist.at[pl.ds(base, sl)])

            @pl.when(sid == 0)
            def _():
                pltpu.sync_copy(degv.at[pl.ds(0, _L)],
                                hist.at[pl.ds(npad, _L)])
            plsc.subcore_barrier()

            e0 = sid * ept

            def batch_body(b, _):
                pltpu.sync_copy(dst_hbm.at[pl.ds(e0 + b * _BATCH, _BATCH)],
                                ibuf)
                pltpu.async_copy(ones, hist.at[ibuf], sem, add=True).wait()
                return 0
            lax.fori_loop(0, nbatch, batch_body, 0)
            if erem:
                for j in range(_BATCH // _L):
                    ibuf[pl.ds(j * _L, _L)] = lane + npad
                pltpu.sync_copy(dst_hbm.at[pl.ds(e0 + nbatch * _BATCH, erem)],
                                ibuf.at[pl.ds(0, erem)])
                pltpu.async_copy(ones, hist.at[ibuf], sem, add=True).wait()
            plsc.subcore_barrier()

            pltpu.sync_copy(hist.at[pl.ds(base, sl)], dfv)
            pltpu.sync_copy(dfv, deg_hbm.at[pl.ds(base, sl)])

    return deg_dinv


def _make_edge_agg(n_nodes, n_edges, chunk):
    npadrows = 128
    cpad = chunk + npadrows
    rpt = chunk // _NTILES
    ept = n_edges // _NTILES
    ngrp = ept // (_NB * _EB)
    tail = ept - ngrp * _NB * _EB
    tfull, trem = tail // _EB, tail % _EB
    nz = rpt // _ZR
    mesh = plsc.VectorSubcoreMesh(core_axis_name="c", subcore_axis_name="s")

    @functools.partial(
        pl.kernel,
        mesh=mesh,
        out_type=jax.ShapeDtypeStruct((4 * chunk, 128), jnp.float32),
        scratch_types=[
            pltpu.VMEM((_EB,), jnp.int32),
            pltpu.VMEM((_NB, _EB), jnp.int32),
            pltpu.VMEM((_NB, _EB), jnp.int32),
            pltpu.VMEM((_NB, _EB, 128), jnp.float32),
            pltpu.VMEM((_ZR, 128), jnp.float32),
            pltpu.VMEM_SHARED((cpad, 128), jnp.float32),
        ] + [pltpu.SemaphoreType.DMA] * (2 * _NB),
    )
    def edge_agg(src_hbm, dst_hbm, y_hbm, s_hbm,
                 dbuf, gsrc, gldst, rowbuf, zbuf,
                 acc, *sems):
        cid = lax.axis_index("c")
        sid = lax.axis_index("s")
        lane = lax.iota(jnp.int32, _L)
        zv = jnp.zeros((_L,), jnp.float32)
        gsem = sems[:_NB]
        ssem = sems[_NB:]

        def zero_zbuf():
            def zrow_body(i, _):
                zbuf[i // 8, pl.ds((i % 8) * _L, _L)] = zv
                return 0
            lax.fori_loop(0, _ZR * 8, zrow_body, 0)
        zero_zbuf()

        e0 = sid * ept
        r0 = sid * rpt

        for p in range(2):
            base = (2 * cid + p) * chunk

            for k in range(nz):
                pltpu.sync_copy(zbuf, acc.at[pl.ds(r0 + k * _ZR, _ZR)])
            plsc.subcore_barrier()

            def clamp_dst(j):
                for g in range(_EB // _L):
                    o = pl.ds(g * _L, _L)
                    off = (j * _EB + g * _L) % npadrows
                    ld = dbuf[o] - base
                    ok = (ld >= 0) & (ld < chunk)
                    gldst[j, o] = jnp.where(ok, ld, chunk + off + lane)

            def stage(j, eoff):
                pltpu.sync_copy(src_hbm.at[pl.ds(eoff, _EB)], gsrc.at[j])
                pltpu.sync_copy(dst_hbm.at[pl.ds(eoff, _EB)], dbuf)
                clamp_dst(j)

            def run_group(eoff, nb):
                gath = []
                for j in range(nb):
                    stage(j, eoff + j * _EB)
                    gath.append(pltpu.async_copy(
                        y_hbm.at[gsrc.at[j]], rowbuf.at[j], gsem[j]))
                scat = []
                for j in range(nb):
                    gath[j].wait()
                    scat.append(pltpu.async_copy(
                        rowbuf.at[j], acc.at[gldst.at[j]], ssem[j],
                        add=True))
                for j in range(nb):
                    scat[j].wait()

            def group_body(k, _):
                run_group(e0 + k * _NB * _EB, _NB)
                return 0
            lax.fori_loop(0, ngrp, group_body, 0)

            toff = e0 + ngrp * _NB * _EB
            if tfull:
                run_group(toff, tfull)
            if trem:
                for g in range(_EB // _L):
                    o = pl.ds(g * _L, _L)
                    gsrc[0, o] = lane
                    dbuf[o] = jnp.full((_L,), -1, jnp.int32)
                eoff = toff + tfull * _EB
                pltpu.sync_copy(src_hbm.at[pl.ds(eoff, trem)],
                                gsrc.at[0, pl.ds(0, trem)])
                pltpu.sync_copy(dst_hbm.at[pl.ds(eoff, trem)],
                                dbuf.at[pl.ds(0, trem)])
                clamp_dst(0)
                pltpu.async_copy(
                    y_hbm.at[gsrc.at[0]], rowbuf.at[0], gsem[0]).wait()
                pltpu.async_copy(
                    rowbuf.at[0], acc.at[gldst.at[0]], ssem[0],
                    add=True).wait()
            plsc.subcore_barrier()

            for k in range(nz):
                pltpu.sync_copy(acc.at[pl.ds(r0 + k * _ZR, _ZR)], zbuf)
                pltpu.sync_copy(
                    zbuf, s_hbm.at[pl.ds(base + r0 + k * _ZR, _ZR)])
            zero_zbuf()
            plsc.subcore_barrier()

    return edge_agg


def _lin_body(x_ref, w1_ref, b1_ref, wg_ref, deg_ref, y_ref):
    h = jnp.maximum(
        jnp.dot(x_ref[...], w1_ref[...],
                preferred_element_type=jnp.float32,
                precision=lax.Precision.HIGHEST) + b1_ref[...], 0.0)
    xw = jnp.dot(h, wg_ref[...], preferred_element_type=jnp.float32,
                 precision=lax.Precision.HIGHEST)
    y_ref[...] = xw * lax.rsqrt(deg_ref[...] + 1.0)


def _head_body(s_ref, y_ref, deg_ref, bg_ref, wb_ref, bc_ref,
               mu_ref, ls_ref):
    dinv = lax.rsqrt(deg_ref[...] + 1.0)
    h = jnp.maximum((s_ref[...] + y_ref[...]) * dinv + bg_ref[...], 0.0)
    prod = jnp.dot(h, wb_ref[...], preferred_element_type=jnp.float32,
                   precision=lax.Precision.HIGHEST) + bc_ref[...]
    mu_ref[...] = prod[:, :128]
    ls_ref[...] = prod[:, 128:]


def kernel(x, edge_index, W1, b1, Wg, bg, Wmu, bmu, Wls, bls):
    n, fin = x.shape
    e = edge_index.shape[1]
    f1 = W1.shape[1]
    f2 = Wg.shape[1]

    sl = (-(-n // _NTILES) + 7) // 8 * 8
    npad = _NTILES * sl
    chunk = (-(-npad // 4) + 255) // 256 * 256
    blk = 1000
    grid = n // blk

    src = edge_index[0]
    dst = edge_index[1]

    deg = _make_deg_dinv(n, e, npad)(dst)

    finp = 32
    xp = jnp.pad(x, ((0, 0), (0, finp - fin)))
    w1p = jnp.pad(W1, ((0, finp - fin), (0, 0)))
    deg2 = deg.reshape(npad, 1)
    y = pl.pallas_call(
        _lin_body,
        grid=(grid,),
        in_specs=[
            pl.BlockSpec((blk, finp), lambda i: (i, 0)),
            pl.BlockSpec((finp, f1), lambda i: (0, 0)),
            pl.BlockSpec((1, f1), lambda i: (0, 0)),
            pl.BlockSpec((f1, f2), lambda i: (0, 0)),
            pl.BlockSpec((blk, 1), lambda i: (i, 0)),
        ],
        out_specs=pl.BlockSpec((blk, f2), lambda i: (i, 0)),
        out_shape=jax.ShapeDtypeStruct((n, f2), jnp.float32),
    )(xp, w1p, b1.reshape(1, f1), Wg, deg2)

    s = _make_edge_agg(n, e, chunk)(src, dst, y)

    half = f2 // 2
    fo = Wmu.shape[1]
    wb = jnp.zeros((f2, 2 * fo), jnp.float32)
    wb = wb.at[:half, :fo].set(Wmu)
    wb = wb.at[half:, fo:].set(Wls)
    bc = jnp.concatenate([bmu, bls]).reshape(1, 2 * fo)
    mu, ls = pl.pallas_call(
        _head_body,
        grid=(grid,),
        in_specs=[
            pl.BlockSpec((blk, f2), lambda i: (i, 0)),
            pl.BlockSpec((blk, f2), lambda i: (i, 0)),
            pl.BlockSpec((blk, 1), lambda i: (i, 0)),
            pl.BlockSpec((1, f2), lambda i: (0, 0)),
            pl.BlockSpec((f2, 2 * fo), lambda i: (0, 0)),
            pl.BlockSpec((1, 2 * fo), lambda i: (0, 0)),
        ],
        out_specs=[
            pl.BlockSpec((blk, fo), lambda i: (i, 0)),
            pl.BlockSpec((blk, fo), lambda i: (i, 0)),
        ],
        out_shape=[
            jax.ShapeDtypeStruct((n, fo), jnp.float32),
            jax.ShapeDtypeStruct((n, fo), jnp.float32),
        ],
    )(s[:n], y, deg2[:n], bg.reshape(1, f2), wb, bc)
    return mu, ls
